# baseline (speedup 1.0000x reference)
import jax
import jax.numpy as jnp
from jax.experimental import pallas as pl
from jax.experimental.pallas import tpu as pltpu

B, T, H, A, E, L = 128, 512, 1024, 128, 16, 1000
NC = 2
EC = E // NC
GE = 4
NG = EC // GE
KC = 256
NK = H // KC


def _adapter_head_kernel(x_ref, vids_ref, bd_ref, bu_ref, bc_ref,
                         Wd_hbm, Wu_hbm, Wc_hbm,
                         out_ref,
                         wd_buf, wu_buf, wc_buf,
                         wd_sem, wu_sem, wc_sem):
    c = pl.program_id(0)
    e0 = c * EC
    for g in range(NG):
        pltpu.make_async_copy(Wd_hbm.at[pl.ds(e0 + g * GE, GE)],
                              wd_buf.at[g], wd_sem.at[g]).start()
        pltpu.make_async_copy(Wu_hbm.at[pl.ds(e0 + g * GE, GE)],
                              wu_buf.at[g], wu_sem.at[g]).start()
    for k in range(NK):
        pltpu.make_async_copy(Wc_hbm.at[pl.ds(k * KC, KC)],
                              wc_buf.at[k], wc_sem.at[k]).start()

    x = x_ref[...]
    vids = vids_ref[...]
    iota = jax.lax.broadcasted_iota(jnp.int32, (B, E), 1)
    onehot = (vids == iota).astype(jnp.float32)
    bdg = jnp.dot(onehot, bd_ref[...],
                  preferred_element_type=jnp.float32)
    bug = jnp.dot(onehot, bu_ref[...], preferred_element_type=jnp.float32)
    act = jnp.where(c == 0, x + bug, jnp.zeros((B, H), jnp.float32))
    for g in range(NG):
        pltpu.make_async_copy(Wd_hbm.at[pl.ds(e0 + g * GE, GE)],
                              wd_buf.at[g], wd_sem.at[g]).wait()
        pltpu.make_async_copy(Wu_hbm.at[pl.ds(e0 + g * GE, GE)],
                              wu_buf.at[g], wu_sem.at[g]).wait()
        for j in range(GE):
            e = e0 + g * GE + j
            m = (vids == e).astype(jnp.float32)
            h = jnp.dot(x, wd_buf[g, j], preferred_element_type=jnp.float32)
            h = jnp.maximum(h + bdg, 0.0) * m
            act = act + jnp.dot(h, wu_buf[g, j],
                                preferred_element_type=jnp.float32)

    acc = jnp.where(c == 0, jnp.broadcast_to(bc_ref[...], (B, L)),
                    jnp.zeros((B, L), jnp.float32))
    for k in range(NK):
        pltpu.make_async_copy(Wc_hbm.at[pl.ds(k * KC, KC)],
                              wc_buf.at[k], wc_sem.at[k]).wait()
        acc = acc + jnp.dot(act[:, k * KC:(k + 1) * KC], wc_buf[k],
                            preferred_element_type=jnp.float32)
    out_ref[0] = acc


def kernel(last_hidden, attention_mask, variety_ids, W_down, b_down, W_up,
           b_up, W_c, b_c):
    x = last_hidden[:, 0, :]
    vids = variety_ids.reshape(B, 1)
    parts = pl.pallas_call(
        _adapter_head_kernel,
        grid=(NC,),
        in_specs=[
            pl.BlockSpec((B, H), lambda c: (0, 0)),
            pl.BlockSpec((B, 1), lambda c: (0, 0)),
            pl.BlockSpec((E, A), lambda c: (0, 0)),
            pl.BlockSpec((E, H), lambda c: (0, 0)),
            pl.BlockSpec((1, L), lambda c: (0, 0)),
            pl.BlockSpec(memory_space=pltpu.MemorySpace.HBM),
            pl.BlockSpec(memory_space=pltpu.MemorySpace.HBM),
            pl.BlockSpec(memory_space=pltpu.MemorySpace.HBM),
        ],
        out_specs=pl.BlockSpec((1, B, L), lambda c: (c, 0, 0)),
        out_shape=jax.ShapeDtypeStruct((NC, B, L), jnp.float32),
        scratch_shapes=[
            pltpu.VMEM((NG, GE, H, A), jnp.float32),
            pltpu.VMEM((NG, GE, A, H), jnp.float32),
            pltpu.VMEM((NK, KC, L), jnp.float32),
            pltpu.SemaphoreType.DMA((NG,)),
            pltpu.SemaphoreType.DMA((NG,)),
            pltpu.SemaphoreType.DMA((NK,)),
        ],
        compiler_params=pltpu.CompilerParams(
            dimension_semantics=("parallel",),
        ),
    )(x, vids, b_down, b_up, b_c.reshape(1, L), W_down, W_up, W_c)
    return parts[0] + parts[1]


# speedup vs baseline: 1.1789x; 1.1789x over previous
import jax
import jax.numpy as jnp
from jax.experimental import pallas as pl
from jax.experimental.pallas import tpu as pltpu

B, T, H, A, E, L = 128, 512, 1024, 128, 16, 1000
GE = 8
NG = E // GE
KC = 512
NK = H // KC


def _adapter_head_kernel(x_ref, vids_ref, bd_ref, bu_ref, bc_ref,
                         Wd_hbm, Wu_hbm, Wc_hbm,
                         out_ref,
                         wd_buf, wu_buf, wc_buf,
                         wd_sem, wu_sem, wc_sem):
    for g in range(NG):
        pltpu.make_async_copy(Wd_hbm.at[pl.ds(g * GE, GE)],
                              wd_buf.at[g], wd_sem.at[g]).start()
        pltpu.make_async_copy(Wu_hbm.at[pl.ds(g * GE, GE)],
                              wu_buf.at[g], wu_sem.at[g]).start()
    for k in range(NK):
        pltpu.make_async_copy(Wc_hbm.at[pl.ds(k * KC, KC)],
                              wc_buf.at[k], wc_sem.at[k]).start()

    x = x_ref[...]
    vids = vids_ref[...]
    iota = jax.lax.broadcasted_iota(jnp.int32, (B, E), 1)
    onehot = (vids == iota).astype(jnp.float32)
    bdg = jnp.dot(onehot, bd_ref[...],
                  preferred_element_type=jnp.float32)
    act = x + jnp.dot(onehot, bu_ref[...],
                      preferred_element_type=jnp.float32)
    for g in range(NG):
        pltpu.make_async_copy(Wd_hbm.at[pl.ds(g * GE, GE)],
                              wd_buf.at[g], wd_sem.at[g]).wait()
        pltpu.make_async_copy(Wu_hbm.at[pl.ds(g * GE, GE)],
                              wu_buf.at[g], wu_sem.at[g]).wait()
        for j in range(GE):
            e = g * GE + j
            m = (vids == e).astype(jnp.float32)
            h = jnp.dot(x, wd_buf[g, j], preferred_element_type=jnp.float32)
            h = jnp.maximum(h + bdg, 0.0) * m
            act = act + jnp.dot(h, wu_buf[g, j],
                                preferred_element_type=jnp.float32)

    acc = jnp.broadcast_to(bc_ref[...], (B, L))
    for k in range(NK):
        pltpu.make_async_copy(Wc_hbm.at[pl.ds(k * KC, KC)],
                              wc_buf.at[k], wc_sem.at[k]).wait()
        acc = acc + jnp.dot(act[:, k * KC:(k + 1) * KC], wc_buf[k],
                            preferred_element_type=jnp.float32)
    out_ref[...] = acc


def kernel(last_hidden, attention_mask, variety_ids, W_down, b_down, W_up,
           b_up, W_c, b_c):
    x = last_hidden[:, 0, :]
    vids = variety_ids.reshape(B, 1)
    logits = pl.pallas_call(
        _adapter_head_kernel,
        grid=(1,),
        in_specs=[
            pl.BlockSpec((B, H), lambda i: (0, 0)),
            pl.BlockSpec((B, 1), lambda i: (0, 0)),
            pl.BlockSpec((E, A), lambda i: (0, 0)),
            pl.BlockSpec((E, H), lambda i: (0, 0)),
            pl.BlockSpec((1, L), lambda i: (0, 0)),
            pl.BlockSpec(memory_space=pltpu.MemorySpace.HBM),
            pl.BlockSpec(memory_space=pltpu.MemorySpace.HBM),
            pl.BlockSpec(memory_space=pltpu.MemorySpace.HBM),
        ],
        out_specs=pl.BlockSpec((B, L), lambda i: (0, 0)),
        out_shape=jax.ShapeDtypeStruct((B, L), jnp.float32),
        scratch_shapes=[
            pltpu.VMEM((NG, GE, H, A), jnp.float32),
            pltpu.VMEM((NG, GE, A, H), jnp.float32),
            pltpu.VMEM((NK, KC, L), jnp.float32),
            pltpu.SemaphoreType.DMA((NG,)),
            pltpu.SemaphoreType.DMA((NG,)),
            pltpu.SemaphoreType.DMA((NK,)),
        ],
    )(x, vids, b_down, b_up, b_c.reshape(1, L), W_down, W_up, W_c)
    return logits


# speedup vs baseline: 1.4219x; 1.2062x over previous
import jax
import jax.numpy as jnp
from jax.experimental import pallas as pl
from jax.experimental.pallas import tpu as pltpu

B, T, H, A, E, L = 128, 512, 1024, 128, 16, 1000
GE = 4
NG = E // GE
KC = 256
NK = H // KC


def _adapter_head_kernel(lh_hbm, vids_ref, bd_ref, bu_ref, bc_ref,
                         Wd_hbm, Wu_hbm, Wc_hbm,
                         out_ref,
                         x_buf, wd_buf, wu_buf, wc_buf,
                         x_sem, wd_sem, wu_sem, wc_sem):
    pltpu.make_async_copy(lh_hbm.at[:, 0], x_buf, x_sem).start()
    for g in range(NG):
        pltpu.make_async_copy(Wd_hbm.at[pl.ds(g * GE, GE)],
                              wd_buf.at[g], wd_sem.at[g]).start()
        pltpu.make_async_copy(Wu_hbm.at[pl.ds(g * GE, GE)],
                              wu_buf.at[g], wu_sem.at[g]).start()
    for k in range(NK):
        pltpu.make_async_copy(Wc_hbm.at[pl.ds(k * KC, KC)],
                              wc_buf.at[k], wc_sem.at[k]).start()

    pltpu.make_async_copy(lh_hbm.at[:, 0], x_buf, x_sem).wait()
    x = x_buf[...]
    vids = vids_ref[...]
    iota = jax.lax.broadcasted_iota(jnp.int32, (B, E), 1)
    onehot = (vids == iota).astype(jnp.float32)
    bdg = jnp.dot(onehot, bd_ref[...],
                  preferred_element_type=jnp.float32)
    act = x + jnp.dot(onehot, bu_ref[...],
                      preferred_element_type=jnp.float32)
    for g in range(NG):
        pltpu.make_async_copy(Wd_hbm.at[pl.ds(g * GE, GE)],
                              wd_buf.at[g], wd_sem.at[g]).wait()
        pltpu.make_async_copy(Wu_hbm.at[pl.ds(g * GE, GE)],
                              wu_buf.at[g], wu_sem.at[g]).wait()
        for j in range(GE):
            e = g * GE + j
            m = (vids == e).astype(jnp.float32)
            h = jnp.dot(x, wd_buf[g, j], preferred_element_type=jnp.float32)
            h = jnp.maximum(h + bdg, 0.0) * m
            act = act + jnp.dot(h, wu_buf[g, j],
                                preferred_element_type=jnp.float32)

    acc = jnp.broadcast_to(bc_ref[...], (B, L))
    for k in range(NK):
        pltpu.make_async_copy(Wc_hbm.at[pl.ds(k * KC, KC)],
                              wc_buf.at[k], wc_sem.at[k]).wait()
        acc = acc + jnp.dot(act[:, k * KC:(k + 1) * KC], wc_buf[k],
                            preferred_element_type=jnp.float32)
    out_ref[...] = acc


def kernel(last_hidden, attention_mask, variety_ids, W_down, b_down, W_up,
           b_up, W_c, b_c):
    vids = variety_ids.reshape(B, 1)
    logits = pl.pallas_call(
        _adapter_head_kernel,
        grid=(1,),
        in_specs=[
            pl.BlockSpec(memory_space=pltpu.MemorySpace.HBM),
            pl.BlockSpec((B, 1), lambda i: (0, 0)),
            pl.BlockSpec((E, A), lambda i: (0, 0)),
            pl.BlockSpec((E, H), lambda i: (0, 0)),
            pl.BlockSpec((1, L), lambda i: (0, 0)),
            pl.BlockSpec(memory_space=pltpu.MemorySpace.HBM),
            pl.BlockSpec(memory_space=pltpu.MemorySpace.HBM),
            pl.BlockSpec(memory_space=pltpu.MemorySpace.HBM),
        ],
        out_specs=pl.BlockSpec((B, L), lambda i: (0, 0)),
        out_shape=jax.ShapeDtypeStruct((B, L), jnp.float32),
        scratch_shapes=[
            pltpu.VMEM((B, H), jnp.float32),
            pltpu.VMEM((NG, GE, H, A), jnp.float32),
            pltpu.VMEM((NG, GE, A, H), jnp.float32),
            pltpu.VMEM((NK, KC, L), jnp.float32),
            pltpu.SemaphoreType.DMA,
            pltpu.SemaphoreType.DMA((NG,)),
            pltpu.SemaphoreType.DMA((NG,)),
            pltpu.SemaphoreType.DMA((NK,)),
        ],
    )(last_hidden, vids, b_down, b_up, b_c.reshape(1, L), W_down, W_up, W_c)
    return logits
